# baseline (speedup 1.0000x reference)
import functools

import jax
import jax.numpy as jnp
import numpy as np
from jax.experimental import pallas as pl
from jax.experimental.pallas import tpu as pltpu

B = 4096
F = 26
D = 16
A = 16
P = F * (F - 1) // 2
PPAD = 384
BB = 128


def _pair_onehots():
    row = np.zeros((F, PPAD), dtype=np.float32)
    col = np.zeros((F, PPAD), dtype=np.float32)
    k = 0
    for i in range(F - 1):
        for j in range(i + 1, F):
            row[i, k] = 1.0
            col[j, k] = 1.0
            k += 1
    return row, col


def _afm_body(emb_t_ref, fv_ref, fow_ref, r_ref, c_ref, pvec_ref,
              attw_ref, attb_ref, atth_ref, bias_ref, out_ref):
    fv = fv_ref[:]
    embv_t = emb_t_ref[:] * fv[:, None, :]
    x = embv_t.reshape(BB * D, F)
    p = jnp.dot(x, r_ref[:], preferred_element_type=jnp.float32)
    q = jnp.dot(x, c_ref[:], preferred_element_type=jnp.float32)
    inter = (p * q).reshape(BB, D, PPAD)

    sig = jnp.zeros((BB, PPAD), dtype=jnp.float32)
    for a in range(A):
        t = jnp.full((BB, PPAD), attb_ref[a], dtype=jnp.float32)
        for d in range(D):
            t = t + inter[:, d, :] * attw_ref[d, a]
        sig = sig + atth_ref[a] * jnp.maximum(t, 0.0)

    lane = jax.lax.broadcasted_iota(jnp.int32, (BB, PPAD), 1)
    sig = jnp.where(lane < P, sig, -1e30)
    m = jnp.max(sig, axis=1, keepdims=True)
    e = jnp.exp(sig - m)
    att = e / jnp.sum(e, axis=1, keepdims=True)

    pool = jnp.sum(att[:, None, :] * inter, axis=2)
    yv = jnp.sum(pool * pvec_ref[:], axis=1)
    y_first = jnp.sum(fow_ref[:] * fv, axis=1)
    y = y_first + yv + bias_ref[0]
    out_ref[:] = 1.0 / (1.0 + jnp.exp(-y))


@functools.partial(jax.jit, static_argnames=())
def _afm_dense(emb_t, fv, fow, r, c, pvec, att_W, att_b, att_h, bias):
    grid = (B // BB,)
    return pl.pallas_call(
        _afm_body,
        grid=grid,
        in_specs=[
            pl.BlockSpec((BB, D, F), lambda i: (i, 0, 0)),
            pl.BlockSpec((BB, F), lambda i: (i, 0)),
            pl.BlockSpec((BB, F), lambda i: (i, 0)),
            pl.BlockSpec((F, PPAD), lambda i: (0, 0)),
            pl.BlockSpec((F, PPAD), lambda i: (0, 0)),
            pl.BlockSpec((1, D), lambda i: (0, 0)),
            pl.BlockSpec(memory_space=pltpu.SMEM),
            pl.BlockSpec(memory_space=pltpu.SMEM),
            pl.BlockSpec(memory_space=pltpu.SMEM),
            pl.BlockSpec(memory_space=pltpu.SMEM),
        ],
        out_specs=pl.BlockSpec((BB,), lambda i: (i,)),
        out_shape=jax.ShapeDtypeStruct((B,), jnp.float32),
    )(emb_t, fv, fow, r, c, pvec, att_W, att_b, att_h, bias)


def kernel(feat_index, feat_value, fo_w, emb_w, att_W, att_b, att_h, p_vec, bias):
    fi = feat_index.astype(jnp.int32)
    emb = jnp.take(emb_w, fi, axis=0)
    emb_t = emb.transpose(0, 2, 1)
    fow = jnp.take(fo_w[:, 0], fi, axis=0)
    r_np, c_np = _pair_onehots()
    r = jnp.asarray(r_np)
    c = jnp.asarray(c_np)
    pvec = p_vec.reshape(1, D)
    return _afm_dense(emb_t, feat_value, fow, r, c, pvec,
                      att_W, att_b, att_h, bias)


# speedup vs baseline: 7.1009x; 7.1009x over previous
import functools

import jax
import jax.numpy as jnp
import numpy as np
from jax.experimental import pallas as pl
from jax.experimental.pallas import tpu as pltpu

B = 4096
F = 26
D = 16
A = 16
P = F * (F - 1) // 2
PPAD = 384
BB = 128


def _pair_onehots():
    row = np.zeros((F, PPAD), dtype=np.float32)
    col = np.zeros((F, PPAD), dtype=np.float32)
    k = 0
    for i in range(F - 1):
        for j in range(i + 1, F):
            row[i, k] = 1.0
            col[j, k] = 1.0
            k += 1
    return row, col


KB = 16
NCHUNK = BB // KB


def _afm_body(emb_t_ref, fv_ref, fow_ref, r_ref, c_ref, pvec_ref,
              kron_ref, btile_ref, htile_ref, bias_ref, out_ref):
    fv = fv_ref[:]
    embv_t = emb_t_ref[:] * fv[:, None, :]
    x = embv_t.reshape(BB * D, F)
    p = jnp.dot(x, r_ref[:], preferred_element_type=jnp.float32)
    q = jnp.dot(x, c_ref[:], preferred_element_type=jnp.float32)
    inter = p * q

    kron = kron_ref[:]
    btile = btile_ref[:]
    htile = htile_ref[:]
    sig_chunks = []
    for cix in range(NCHUNK):
        chunk = inter[cix * KB * D:(cix + 1) * KB * D, :]
        t = jnp.dot(kron, chunk, preferred_element_type=jnp.float32)
        r = htile * jnp.maximum(t + btile, 0.0)
        sig_chunks.append(jnp.sum(r.reshape(KB, A, PPAD), axis=1))
    sig = jnp.concatenate(sig_chunks, axis=0)

    lane = jax.lax.broadcasted_iota(jnp.int32, (BB, PPAD), 1)
    sig = jnp.where(lane < P, sig, -1e30)
    m = jnp.max(sig, axis=1, keepdims=True)
    e = jnp.exp(sig - m)
    att = e / jnp.sum(e, axis=1, keepdims=True)

    inter3 = inter.reshape(BB, D, PPAD)
    pool = jnp.sum(att[:, None, :] * inter3, axis=2)
    yv = jnp.sum(pool * pvec_ref[:], axis=1)
    y_first = jnp.sum(fow_ref[:] * fv, axis=1)
    y = y_first + yv + bias_ref[0]
    out_ref[:] = 1.0 / (1.0 + jnp.exp(-y))


@functools.partial(jax.jit, static_argnames=())
def _afm_dense(emb_t, fv, fow, r, c, pvec, kron, btile, htile, bias):
    grid = (B // BB,)
    return pl.pallas_call(
        _afm_body,
        grid=grid,
        in_specs=[
            pl.BlockSpec((BB, D, F), lambda i: (i, 0, 0)),
            pl.BlockSpec((BB, F), lambda i: (i, 0)),
            pl.BlockSpec((BB, F), lambda i: (i, 0)),
            pl.BlockSpec((F, PPAD), lambda i: (0, 0)),
            pl.BlockSpec((F, PPAD), lambda i: (0, 0)),
            pl.BlockSpec((1, D), lambda i: (0, 0)),
            pl.BlockSpec((KB * A, KB * D), lambda i: (0, 0)),
            pl.BlockSpec((KB * A, 1), lambda i: (0, 0)),
            pl.BlockSpec((KB * A, 1), lambda i: (0, 0)),
            pl.BlockSpec(memory_space=pltpu.SMEM),
        ],
        out_specs=pl.BlockSpec((BB,), lambda i: (i,)),
        out_shape=jax.ShapeDtypeStruct((B,), jnp.float32),
    )(emb_t, fv, fow, r, c, pvec, kron, btile, htile, bias)


def kernel(feat_index, feat_value, fo_w, emb_w, att_W, att_b, att_h, p_vec, bias):
    fi = feat_index.astype(jnp.int32)
    emb = jnp.take(emb_w, fi, axis=0)
    emb_t = emb.transpose(0, 2, 1)
    fow = jnp.take(fo_w[:, 0], fi, axis=0)
    r_np, c_np = _pair_onehots()
    r = jnp.asarray(r_np)
    c = jnp.asarray(c_np)
    pvec = p_vec.reshape(1, D)
    kron = jnp.kron(jnp.eye(KB, dtype=jnp.float32), att_W.T)
    btile = jnp.tile(att_b, KB).reshape(KB * A, 1)
    htile = jnp.tile(att_h, KB).reshape(KB * A, 1)
    return _afm_dense(emb_t, feat_value, fow, r, c, pvec,
                      kron, btile, htile, bias)
